# baseline (speedup 1.0000x reference)
import functools

import jax
import jax.numpy as jnp
from jax import lax
from jax.experimental import pallas as pl
from jax.experimental.pallas import tpu as pltpu
from jax.experimental.pallas import tpu_sc as plsc

_NUM_CORES = 2
_NUM_SUBCORES = 16
_NUM_WORKERS = _NUM_CORES * _NUM_SUBCORES
_LANES = 16
_CHUNK = 16000
_UNROLL = 8


def _sc_sse_body(p_hbm, t_hbm, o_hbm, pb, tb, ob, sem_p0, sem_p1, sem_t0, sem_t1):
    wid = lax.axis_index("s") * _NUM_CORES + lax.axis_index("c")
    n_total = p_hbm.shape[0]
    per_worker = n_total // _NUM_WORKERS
    n_chunks = per_worker // _CHUNK
    base = wid * per_worker

    sem_p = (sem_p0, sem_p1)
    sem_t = (sem_t0, sem_t1)

    def copies(k, slot):
        src = pl.ds(base + k * _CHUNK, _CHUNK)
        cp = pltpu.make_async_copy(p_hbm.at[src], pb.at[slot], sem_p[slot])
        ct = pltpu.make_async_copy(t_hbm.at[src], tb.at[slot], sem_t[slot])
        return cp, ct

    cp, ct = copies(0, 0)
    cp.start()
    ct.start()

    acc = tuple(jnp.zeros((_LANES,), jnp.float32) for _ in range(_UNROLL))
    for k in range(n_chunks):
        slot = k % 2
        if k + 1 < n_chunks:
            cpn, ctn = copies(k + 1, 1 - slot)
            cpn.start()
            ctn.start()
        cp, ct = copies(k, slot)
        cp.wait()
        ct.wait()

        pb_s = pb.at[slot]
        tb_s = tb.at[slot]

        def body(j, accs):
            group = j * (_UNROLL * _LANES)
            out = []
            for u in range(_UNROLL):
                off = group + u * _LANES
                d = pb_s[pl.ds(off, _LANES)] - tb_s[pl.ds(off, _LANES)]
                out.append(accs[u] + d * d)
            return tuple(out)

        acc = lax.fori_loop(0, _CHUNK // (_UNROLL * _LANES), body, acc)

    total = acc[0]
    for u in range(1, _UNROLL):
        total = total + acc[u]
    ob[...] = total
    pltpu.sync_copy(ob, o_hbm.at[wid])


def _sc_partial_sums(pred_flat, target_flat):
    mesh = plsc.VectorSubcoreMesh(core_axis_name="c", subcore_axis_name="s")
    kern = functools.partial(
        pl.kernel,
        mesh=mesh,
        out_type=jax.ShapeDtypeStruct((_NUM_WORKERS, _LANES), jnp.float32),
        scratch_types=[
            pltpu.VMEM((2, _CHUNK), jnp.float32),
            pltpu.VMEM((2, _CHUNK), jnp.float32),
            pltpu.VMEM((_LANES,), jnp.float32),
            pltpu.SemaphoreType.DMA,
            pltpu.SemaphoreType.DMA,
            pltpu.SemaphoreType.DMA,
            pltpu.SemaphoreType.DMA,
        ],
    )(_sc_sse_body)
    return kern(pred_flat, target_flat)


def kernel(pred, target, batch_idx, num_graphs):
    del batch_idx
    n_rows, n_feat = pred.shape
    partials = _sc_partial_sums(
        pred.reshape(n_rows * n_feat), target.reshape(n_rows * n_feat)
    )
    return jnp.sum(partials) / num_graphs


# speedup vs baseline: 1.6561x; 1.6561x over previous
import functools

import jax
import jax.numpy as jnp
from jax import lax
from jax.experimental import pallas as pl
from jax.experimental.pallas import tpu as pltpu
from jax.experimental.pallas import tpu_sc as plsc

_SC_ROWS = 25600
_TC_BLOCK_ROWS = 7440

_NUM_CORES = 2
_NUM_SUBCORES = 16
_NUM_WORKERS = _NUM_CORES * _NUM_SUBCORES
_LANES = 16
_SC_CHUNK = 12800
_UNROLL = 8


def _tc_sse_kernel(ng_ref, p_ref, t_ref, o_ref):
    i = pl.program_id(0)

    @pl.when(i == 0)
    def _():
        o_ref[0] = 0.0

    d = p_ref[...] - t_ref[...]
    o_ref[0] += jnp.sum(d * d)

    @pl.when(i == pl.num_programs(0) - 1)
    def _():
        o_ref[0] = o_ref[0] / ng_ref[0]


def _sc_sse_body(base_elt, n_elts, p_hbm, t_hbm, o_hbm, pb, tb, ob,
                 sem_p0, sem_p1, sem_t0, sem_t1):
    wid = lax.axis_index("s") * _NUM_CORES + lax.axis_index("c")
    per_worker = n_elts // _NUM_WORKERS
    n_chunks = per_worker // _SC_CHUNK
    base = base_elt + wid * per_worker

    sem_p = (sem_p0, sem_p1)
    sem_t = (sem_t0, sem_t1)

    def copies(k, slot):
        src = pl.ds(base + k * _SC_CHUNK, _SC_CHUNK)
        cp = pltpu.make_async_copy(p_hbm.at[src], pb.at[slot], sem_p[slot])
        ct = pltpu.make_async_copy(t_hbm.at[src], tb.at[slot], sem_t[slot])
        return cp, ct

    cp, ct = copies(0, 0)
    cp.start()
    ct.start()

    acc = tuple(jnp.zeros((_LANES,), jnp.float32) for _ in range(_UNROLL))
    for k in range(n_chunks):
        slot = k % 2
        if k + 1 < n_chunks:
            cpn, ctn = copies(k + 1, 1 - slot)
            cpn.start()
            ctn.start()
        cp, ct = copies(k, slot)
        cp.wait()
        ct.wait()

        pb_s = pb.at[slot]
        tb_s = tb.at[slot]

        def body(j, accs):
            group = j * (_UNROLL * _LANES)
            out = []
            for u in range(_UNROLL):
                off = group + u * _LANES
                d = pb_s[pl.ds(off, _LANES)] - tb_s[pl.ds(off, _LANES)]
                out.append(accs[u] + d * d)
            return tuple(out)

        acc = lax.fori_loop(0, _SC_CHUNK // (_UNROLL * _LANES), body, acc)

    total = acc[0]
    for u in range(1, _UNROLL):
        total = total + acc[u]
    ob[...] = total
    pltpu.sync_copy(ob, o_hbm.at[wid])


def _sc_partial_sums(pred_flat, target_flat, base_elt, n_elts):
    mesh = plsc.VectorSubcoreMesh(core_axis_name="c", subcore_axis_name="s")
    kern = functools.partial(
        pl.kernel,
        mesh=mesh,
        out_type=jax.ShapeDtypeStruct((_NUM_WORKERS, _LANES), jnp.float32),
        scratch_types=[
            pltpu.VMEM((2, _SC_CHUNK), jnp.float32),
            pltpu.VMEM((2, _SC_CHUNK), jnp.float32),
            pltpu.VMEM((_LANES,), jnp.float32),
            pltpu.SemaphoreType.DMA,
            pltpu.SemaphoreType.DMA,
            pltpu.SemaphoreType.DMA,
            pltpu.SemaphoreType.DMA,
        ],
    )(functools.partial(_sc_sse_body, base_elt, n_elts))
    return kern(pred_flat, target_flat)


def kernel(pred, target, batch_idx, num_graphs):
    del batch_idx
    n_rows, n_feat = pred.shape
    tc_rows = n_rows - _SC_ROWS
    ng = jnp.asarray(num_graphs, jnp.float32).reshape(1)

    tc_part = pl.pallas_call(
        _tc_sse_kernel,
        grid=(tc_rows // _TC_BLOCK_ROWS,),
        in_specs=[
            pl.BlockSpec(memory_space=pltpu.SMEM),
            pl.BlockSpec((_TC_BLOCK_ROWS, n_feat), lambda i: (i, 0)),
            pl.BlockSpec((_TC_BLOCK_ROWS, n_feat), lambda i: (i, 0)),
        ],
        out_specs=pl.BlockSpec((1,), lambda i: (0,), memory_space=pltpu.SMEM),
        out_shape=jax.ShapeDtypeStruct((1,), jnp.float32),
    )(ng, pred, target)

    sc_partials = _sc_partial_sums(
        pred.reshape(n_rows * n_feat),
        target.reshape(n_rows * n_feat),
        tc_rows * n_feat,
        _SC_ROWS * n_feat,
    )
    return tc_part[0] + jnp.sum(sc_partials) / num_graphs


# speedup vs baseline: 1.6566x; 1.0003x over previous
import functools

import jax
import jax.numpy as jnp
from jax import lax
from jax.experimental import pallas as pl
from jax.experimental.pallas import tpu as pltpu
from jax.experimental.pallas import tpu_sc as plsc

_SC_ROWS = 25600
_TC_BLOCK_ROWS = 7440

_NUM_CORES = 2
_NUM_SUBCORES = 16
_NUM_WORKERS = _NUM_CORES * _NUM_SUBCORES
_LANES = 16
_SC_CHUNK = 12800
_UNROLL = 8


def _tc_sse_kernel(ng_ref, p_ref, t_ref, o_ref):
    i = pl.program_id(0)

    @pl.when(i == 0)
    def _():
        o_ref[0] = 0.0

    d = p_ref[...] - t_ref[...]
    o_ref[0] += jnp.sum(d * d)

    @pl.when(i == pl.num_programs(0) - 1)
    def _():
        o_ref[0] = o_ref[0] / ng_ref[0]


def _sc_sse_body(base_elt, n_elts, p_hbm, t_hbm, o_hbm, pb, tb, ob,
                 sem_p0, sem_p1, sem_t0, sem_t1):
    wid = lax.axis_index("s") * _NUM_CORES + lax.axis_index("c")
    per_worker = n_elts // _NUM_WORKERS
    n_chunks = per_worker // _SC_CHUNK
    base = base_elt + wid * per_worker

    sem_p = (sem_p0, sem_p1)
    sem_t = (sem_t0, sem_t1)

    def copies(k, slot):
        src = pl.ds(base + k * _SC_CHUNK, _SC_CHUNK)
        cp = pltpu.make_async_copy(p_hbm.at[src], pb.at[slot], sem_p[slot])
        ct = pltpu.make_async_copy(t_hbm.at[src], tb.at[slot], sem_t[slot])
        return cp, ct

    cp, ct = copies(0, 0)
    cp.start()
    ct.start()

    acc = tuple(jnp.zeros((_LANES,), jnp.float32) for _ in range(_UNROLL))
    for k in range(n_chunks):
        slot = k % 2
        if k + 1 < n_chunks:
            cpn, ctn = copies(k + 1, 1 - slot)
            cpn.start()
            ctn.start()
        cp, ct = copies(k, slot)
        cp.wait()
        ct.wait()

        pb_s = pb.at[slot]
        tb_s = tb.at[slot]

        def body(j, accs):
            group = j * (_UNROLL * _LANES)
            out = []
            for u in range(_UNROLL):
                off = group + u * _LANES
                d = pb_s[pl.ds(off, _LANES)] - tb_s[pl.ds(off, _LANES)]
                out.append(accs[u] + d * d)
            return tuple(out)

        acc = lax.fori_loop(0, _SC_CHUNK // (_UNROLL * _LANES), body, acc)

    total = acc[0]
    for u in range(1, _UNROLL):
        total = total + acc[u]
    ob[...] = total
    pltpu.sync_copy(ob, o_hbm.at[wid])


def _sc_partial_sums(pred_flat, target_flat, base_elt, n_elts):
    mesh = plsc.VectorSubcoreMesh(core_axis_name="c", subcore_axis_name="s", num_cores=2)
    kern = functools.partial(
        pl.kernel,
        mesh=mesh,
        out_type=jax.ShapeDtypeStruct((_NUM_WORKERS, _LANES), jnp.float32),
        scratch_types=[
            pltpu.VMEM((2, _SC_CHUNK), jnp.float32),
            pltpu.VMEM((2, _SC_CHUNK), jnp.float32),
            pltpu.VMEM((_LANES,), jnp.float32),
            pltpu.SemaphoreType.DMA,
            pltpu.SemaphoreType.DMA,
            pltpu.SemaphoreType.DMA,
            pltpu.SemaphoreType.DMA,
        ],
    )(functools.partial(_sc_sse_body, base_elt, n_elts))
    return kern(pred_flat, target_flat)


def kernel(pred, target, batch_idx, num_graphs):
    del batch_idx
    n_rows, n_feat = pred.shape
    tc_rows = n_rows - _SC_ROWS
    ng = jnp.asarray(num_graphs, jnp.float32).reshape(1)

    tc_part = pl.pallas_call(
        _tc_sse_kernel,
        grid=(tc_rows // _TC_BLOCK_ROWS,),
        in_specs=[
            pl.BlockSpec(memory_space=pltpu.SMEM),
            pl.BlockSpec((_TC_BLOCK_ROWS, n_feat), lambda i: (i, 0)),
            pl.BlockSpec((_TC_BLOCK_ROWS, n_feat), lambda i: (i, 0)),
        ],
        out_specs=pl.BlockSpec((1,), lambda i: (0,), memory_space=pltpu.SMEM),
        out_shape=jax.ShapeDtypeStruct((1,), jnp.float32),
    )(ng, pred, target)

    sc_partials = _sc_partial_sums(
        pred.reshape(n_rows * n_feat),
        target.reshape(n_rows * n_feat),
        tc_rows * n_feat,
        _SC_ROWS * n_feat,
    )
    return tc_part[0] + jnp.sum(sc_partials) / num_graphs


# speedup vs baseline: 1.6762x; 1.0118x over previous
import functools

import jax
import jax.numpy as jnp
from jax import lax
from jax.experimental import pallas as pl
from jax.experimental.pallas import tpu as pltpu
from jax.experimental.pallas import tpu_sc as plsc

_SC_ROWS = 9600
_TC_BLOCK_ROWS = 9040

_NUM_CORES = 2
_NUM_SUBCORES = 16
_NUM_WORKERS = _NUM_CORES * _NUM_SUBCORES
_LANES = 16
_SC_CHUNK = 12800
_UNROLL = 8


def _tc_sse_kernel(ng_ref, p_ref, t_ref, o_ref):
    i = pl.program_id(0)

    @pl.when(i == 0)
    def _():
        o_ref[0] = 0.0

    d = p_ref[...] - t_ref[...]
    o_ref[0] += jnp.sum(d * d)

    @pl.when(i == pl.num_programs(0) - 1)
    def _():
        o_ref[0] = o_ref[0] / ng_ref[0]


def _sc_sse_body(base_elt, n_elts, p_hbm, t_hbm, o_hbm, pb, tb, ob,
                 sem_p0, sem_p1, sem_t0, sem_t1):
    wid = lax.axis_index("s") * _NUM_CORES + lax.axis_index("c")
    per_worker = n_elts // _NUM_WORKERS
    n_chunks = per_worker // _SC_CHUNK
    base = base_elt + wid * per_worker

    sem_p = (sem_p0, sem_p1)
    sem_t = (sem_t0, sem_t1)

    def copies(k, slot):
        src = pl.ds(base + k * _SC_CHUNK, _SC_CHUNK)
        cp = pltpu.make_async_copy(p_hbm.at[src], pb.at[slot], sem_p[slot])
        ct = pltpu.make_async_copy(t_hbm.at[src], tb.at[slot], sem_t[slot])
        return cp, ct

    cp, ct = copies(0, 0)
    cp.start()
    ct.start()

    acc = tuple(jnp.zeros((_LANES,), jnp.float32) for _ in range(_UNROLL))
    for k in range(n_chunks):
        slot = k % 2
        if k + 1 < n_chunks:
            cpn, ctn = copies(k + 1, 1 - slot)
            cpn.start()
            ctn.start()
        cp, ct = copies(k, slot)
        cp.wait()
        ct.wait()

        pb_s = pb.at[slot]
        tb_s = tb.at[slot]

        def body(j, accs):
            group = j * (_UNROLL * _LANES)
            out = []
            for u in range(_UNROLL):
                off = group + u * _LANES
                d = pb_s[pl.ds(off, _LANES)] - tb_s[pl.ds(off, _LANES)]
                out.append(accs[u] + d * d)
            return tuple(out)

        acc = lax.fori_loop(0, _SC_CHUNK // (_UNROLL * _LANES), body, acc)

    total = acc[0]
    for u in range(1, _UNROLL):
        total = total + acc[u]
    ob[...] = total
    pltpu.sync_copy(ob, o_hbm.at[wid])


def _sc_partial_sums(pred_flat, target_flat, base_elt, n_elts):
    mesh = plsc.VectorSubcoreMesh(core_axis_name="c", subcore_axis_name="s", num_cores=2)
    kern = functools.partial(
        pl.kernel,
        mesh=mesh,
        out_type=jax.ShapeDtypeStruct((_NUM_WORKERS, _LANES), jnp.float32),
        scratch_types=[
            pltpu.VMEM((2, _SC_CHUNK), jnp.float32),
            pltpu.VMEM((2, _SC_CHUNK), jnp.float32),
            pltpu.VMEM((_LANES,), jnp.float32),
            pltpu.SemaphoreType.DMA,
            pltpu.SemaphoreType.DMA,
            pltpu.SemaphoreType.DMA,
            pltpu.SemaphoreType.DMA,
        ],
    )(functools.partial(_sc_sse_body, base_elt, n_elts))
    return kern(pred_flat, target_flat)


def kernel(pred, target, batch_idx, num_graphs):
    del batch_idx
    n_rows, n_feat = pred.shape
    tc_rows = n_rows - _SC_ROWS
    ng = jnp.asarray(num_graphs, jnp.float32).reshape(1)

    sc_partials = _sc_partial_sums(
        pred.reshape(n_rows * n_feat),
        target.reshape(n_rows * n_feat),
        tc_rows * n_feat,
        _SC_ROWS * n_feat,
    )

    tc_part = pl.pallas_call(
        _tc_sse_kernel,
        grid=(tc_rows // _TC_BLOCK_ROWS,),
        in_specs=[
            pl.BlockSpec(memory_space=pltpu.SMEM),
            pl.BlockSpec((_TC_BLOCK_ROWS, n_feat), lambda i: (i, 0)),
            pl.BlockSpec((_TC_BLOCK_ROWS, n_feat), lambda i: (i, 0)),
        ],
        out_specs=pl.BlockSpec((1,), lambda i: (0,), memory_space=pltpu.SMEM),
        out_shape=jax.ShapeDtypeStruct((1,), jnp.float32),
    )(ng, pred, target)

    return tc_part[0] + jnp.sum(sc_partials) / num_graphs


# speedup vs baseline: 2.6552x; 1.5841x over previous
import jax
import jax.numpy as jnp
from jax.experimental import pallas as pl
from jax.experimental.pallas import tpu as pltpu

_BLOCK_ROWS = 5000


def _sse_block_kernel(ng_ref, pa_ref, pb_ref, ta_ref, tb_ref, o_ref):
    i = pl.program_id(0)

    @pl.when(i == 0)
    def _():
        o_ref[0] = 0.0

    da = pa_ref[...] - ta_ref[...]
    db = pb_ref[...] - tb_ref[...]
    o_ref[0] += jnp.sum(da * da) + jnp.sum(db * db)

    @pl.when(i == pl.num_programs(0) - 1)
    def _():
        o_ref[0] = o_ref[0] / ng_ref[0]


def kernel(pred, target, batch_idx, num_graphs):
    del batch_idx
    n_rows, n_feat = pred.shape
    ng = jnp.asarray(num_graphs, jnp.float32).reshape(1)
    n_blocks = n_rows // _BLOCK_ROWS
    half = n_blocks // 2
    total = pl.pallas_call(
        _sse_block_kernel,
        grid=(half,),
        in_specs=[
            pl.BlockSpec(memory_space=pltpu.SMEM),
            pl.BlockSpec((_BLOCK_ROWS, n_feat), lambda i: (i, 0)),
            pl.BlockSpec((_BLOCK_ROWS, n_feat), lambda i, h=half: (h + i, 0)),
            pl.BlockSpec((_BLOCK_ROWS, n_feat), lambda i: (i, 0)),
            pl.BlockSpec((_BLOCK_ROWS, n_feat), lambda i, h=half: (h + i, 0)),
        ],
        out_specs=pl.BlockSpec(
            (1,), lambda i: (0,), memory_space=pltpu.SMEM
        ),
        out_shape=jax.ShapeDtypeStruct((1,), jnp.float32),
    )(ng, pred, pred, target, target)
    return total[0]
